# baseline (speedup 1.0000x reference)
import functools

import jax
import jax.numpy as jnp
from jax import lax
from jax.experimental import pallas as pl
from jax.experimental.pallas import tpu as pltpu
from jax.experimental.pallas import tpu_sc as plsc

_N = 10000
_E = 160000
_D = 256
_H = 256
_C = 10
_HALF = 128
_CHUNK = 128
_NSUB = 16
_NCHUNKS = _E // _CHUNK
_NPAD = 10112
_TILE_ROWS = _NPAD // _NSUB
_NBUF = 3
_RB = 1000


def _sc_aggregate(hn2, src2d, dst2d):
  mesh = plsc.VectorSubcoreMesh(core_axis_name="c", subcore_axis_name="s")

  @functools.partial(
      pl.kernel,
      out_type=jax.ShapeDtypeStruct((2 * _NPAD, _HALF), jnp.float32),
      mesh=mesh,
      scratch_types=[
          [pltpu.VMEM((_CHUNK,), jnp.int32)] * _NBUF,
          [pltpu.VMEM((_CHUNK,), jnp.int32)] * _NBUF,
          [pltpu.VMEM((_CHUNK, _HALF), jnp.float32)] * _NBUF,
          pltpu.VMEM_SHARED((_NPAD, _HALF), jnp.float32),
          [pltpu.SemaphoreType.DMA] * _NBUF,
          [pltpu.SemaphoreType.DMA] * _NBUF,
      ],
  )
  def agg_kernel(hn2_hbm, src_hbm, dst_hbm, out_hbm, gidx, didx, bufs,
                 acc_sh, gsem, ssem):
    cid = lax.axis_index("c")
    sid = lax.axis_index("s")
    row_off = cid * _NPAD

    buf0 = bufs[0]

    def _zero_row(r, _):
      for j in range(_HALF // 16):
        buf0[r, pl.ds(j * 16, 16)] = jnp.zeros((16,), jnp.float32)
      return 0
    lax.fori_loop(0, _CHUNK, _zero_row, 0)
    for q in range(4):
      pltpu.sync_copy(buf0,
                      acc_sh.at[pl.ds(sid * _TILE_ROWS + q * _CHUNK, _CHUNK)])
    pltpu.sync_copy(
        buf0.at[pl.ds(0, _TILE_ROWS - 4 * _CHUNK)],
        acc_sh.at[pl.ds(sid * _TILE_ROWS + 4 * _CHUNK,
                        _TILE_ROWS - 4 * _CHUNK)])
    plsc.subcore_barrier()

    def _stage(j, m):
      pltpu.sync_copy(src_hbm.at[j], gidx[m])
      pltpu.sync_copy(dst_hbm.at[j], didx[m])
      for i in range(_CHUNK // 16):
        sl = pl.ds(i * 16, 16)
        gidx[m][sl] = gidx[m][sl] + row_off

    def _gather(m):
      pltpu.async_copy(hn2_hbm.at[gidx[m]], bufs[m], gsem[m])

    def _gwait(m):
      pltpu.make_async_copy(hn2_hbm.at[gidx[m]], bufs[m], gsem[m]).wait()

    def _scatter(m):
      pltpu.async_copy(bufs[m], acc_sh.at[didx[m]], ssem[m], add=True)

    def _swait(m):
      pltpu.make_async_copy(bufs[m], acc_sh.at[didx[m]], ssem[m]).wait()

    for m in range(_NBUF):
      _stage(sid + m * _NSUB, m)
      _gather(m)

    n_groups = (_NCHUNKS // _NSUB + _NBUF) // _NBUF

    def _group(k, _):
      for m in range(_NBUF):
        t = _NBUF * k + m
        j = sid + t * _NSUB

        @pl.when(j < _NCHUNKS)
        def _():
          _gwait(m)
          _scatter(m)

        jn = j + _NBUF * _NSUB

        @pl.when(jn < _NCHUNKS)
        def _():
          _swait(m)
          _stage(jn, m)
          _gather(m)
      return 0

    lax.fori_loop(0, n_groups, _group, 0)
    for m in range(_NBUF):
      _swait(m)
    plsc.subcore_barrier()

    pltpu.sync_copy(
        acc_sh.at[pl.ds(sid * _TILE_ROWS, _TILE_ROWS)],
        out_hbm.at[pl.ds(row_off + sid * _TILE_ROWS, _TILE_ROWS)])

  return agg_kernel(hn2, src2d, dst2d)


def _tc_layer(h_or_self, agg, Ws, Wn, b):
  first = agg is None

  def body(*refs):
    if first:
      x_ref, ws_ref, wn_ref, b_ref, self_ref, hn2_ref = refs
      h = x_ref[...]
    else:
      s_ref, agg_ref, ws_ref, wn_ref, b_ref, self_ref, hn2_ref = refs
      h = jnp.maximum(
          s_ref[...] + jnp.concatenate([agg_ref[0], agg_ref[1]], axis=1), 0.0)
    self_ref[...] = (
        jnp.dot(h, ws_ref[...], preferred_element_type=jnp.float32) + b_ref[...])
    hn = jnp.dot(h, wn_ref[...], preferred_element_type=jnp.float32)
    hn2_ref[0] = hn[:, :_HALF]
    hn2_ref[1] = hn[:, _HALF:]

  in_specs = [pl.BlockSpec((_RB, _D), lambda i: (i, 0))]
  operands = [h_or_self]
  if not first:
    in_specs.append(pl.BlockSpec((2, _RB, _HALF), lambda i: (0, i, 0)))
    operands.append(agg.reshape(2, _NPAD, _HALF))
  in_specs += [
      pl.BlockSpec((_D, _H), lambda i: (0, 0)),
      pl.BlockSpec((_D, _H), lambda i: (0, 0)),
      pl.BlockSpec((1, _H), lambda i: (0, 0)),
  ]
  operands += [Ws, Wn, b.reshape(1, _H)]

  self_out, hn2 = pl.pallas_call(
      body,
      grid=(_N // _RB,),
      in_specs=in_specs,
      out_specs=[
          pl.BlockSpec((_RB, _H), lambda i: (i, 0)),
          pl.BlockSpec((2, _RB, _HALF), lambda i: (0, i, 0)),
      ],
      out_shape=[
          jax.ShapeDtypeStruct((_N, _H), jnp.float32),
          jax.ShapeDtypeStruct((2, _NPAD, _HALF), jnp.float32),
      ],
  )(*operands)
  return self_out, hn2.reshape(2 * _NPAD, _HALF)


def _tc_head(self_prev, agg, W1, b1, W2, b2):
  def body(s_ref, agg_ref, w1_ref, b1_ref, w2_ref, b2_ref, out_ref):
    h = jnp.maximum(
        s_ref[...] + jnp.concatenate([agg_ref[0], agg_ref[1]], axis=1), 0.0)
    t = jnp.dot(h, w1_ref[...], preferred_element_type=jnp.float32) + b1_ref[...]
    logits = (jnp.dot(t, w2_ref[...], preferred_element_type=jnp.float32)
              + b2_ref[...])
    m = jnp.max(logits, axis=1, keepdims=True)
    z = logits - m
    out_ref[...] = z - jnp.log(jnp.sum(jnp.exp(z), axis=1, keepdims=True))

  return pl.pallas_call(
      body,
      grid=(_N // _RB,),
      in_specs=[
          pl.BlockSpec((_RB, _H), lambda i: (i, 0)),
          pl.BlockSpec((2, _RB, _HALF), lambda i: (0, i, 0)),
          pl.BlockSpec((_H, _H), lambda i: (0, 0)),
          pl.BlockSpec((1, _H), lambda i: (0, 0)),
          pl.BlockSpec((_H, _C), lambda i: (0, 0)),
          pl.BlockSpec((1, _C), lambda i: (0, 0)),
      ],
      out_specs=pl.BlockSpec((_RB, _C), lambda i: (i, 0)),
      out_shape=jax.ShapeDtypeStruct((_N, _C), jnp.float32),
  )(self_prev, agg.reshape(2, _NPAD, _HALF), W1, b1.reshape(1, _H), W2,
    b2.reshape(1, _C))


def kernel(x, edge_index, W_self_0, W_neigh_0, b_0, W_self_1, W_neigh_1, b_1,
           W_self_2, W_neigh_2, b_2, W_self_3, W_neigh_3, b_3, W1, b1, W2, b2):
  src2d = edge_index[0].reshape(_NCHUNKS, _CHUNK)
  dst2d = edge_index[1].reshape(_NCHUNKS, _CHUNK)

  layers = [(W_self_0, W_neigh_0, b_0), (W_self_1, W_neigh_1, b_1),
            (W_self_2, W_neigh_2, b_2), (W_self_3, W_neigh_3, b_3)]

  self_h, hn2 = _tc_layer(x, None, *layers[0])
  agg = _sc_aggregate(hn2, src2d, dst2d)
  for Ws, Wn, b in layers[1:]:
    self_h, hn2 = _tc_layer(self_h, agg, Ws, Wn, b)
    agg = _sc_aggregate(hn2, src2d, dst2d)
  return _tc_head(self_h, agg, W1, b1, W2, b2)
